# Initial kernel scaffold; baseline (speedup 1.0000x reference)
import jax
import jax.numpy as jnp
from jax.experimental import pallas as pl

_NT, _NY, _NX = 5, 32, 32
_NB = _NY * _NX
_D9 = [(dy, dx) for dy in (-1, 0, 1) for dx in (-1, 0, 1)]
_F25 = [(fy, fx) for fy in (-2, -1, 0, 1, 2) for fx in (-2, -1, 0, 1, 2)]


def _softplus10(z):
    return jax.nn.softplus(10.0 * z) / 10.0


def _encoder_body(x_ref, w1_ref, w2_ref, w3_ref, out_ref):
    lane = jax.lax.broadcasted_iota(jnp.int32, (1, _NB), 1)
    yy = lane // _NX
    xx = lane % _NX

    def conv(h, w_ref):
        acc = None
        for t, (dy, dx) in enumerate(_D9):
            s = dy * _NX + dx
            rolled = jnp.roll(h, -s, axis=1)
            m = ((yy + dy >= 0) & (yy + dy < _NY)
                 & (xx + dx >= 0) & (xx + dx < _NX))
            hs = jnp.where(m, rolled, 0.0)
            p = jnp.dot(w_ref[t], hs, preferred_element_type=jnp.float32)
            acc = p if acc is None else acc + p
        return acc

    h = conv(jax.nn.relu(x_ref[...]), w1_ref)
    h = conv(jax.nn.relu(h), w2_ref)
    out_ref[...] = conv(h, w3_ref)


def _coeff_body(ks_ref, m1_ref, m2_ref, ga_ref, vx_ref, vy_ref, g_ref, c_ref):
    lane9 = jax.lax.broadcasted_iota(jnp.int32, (9, _NB), 1)
    xn = lane9 % _NX
    for k in range(_NT):
        kap = _softplus10(ks_ref[k:k + 1, :])
        gam = _softplus10(ga_ref[k:k + 1, :])
        vxk = vx_ref[k:k + 1, :]
        vyk = vy_ref[k:k + 1, :]
        m1 = m1_ref[k:k + 1, :]
        m2 = m2_ref[k:k + 1, :]
        a = gam + vxk * vxk
        bb = vxk * vyk
        cc = gam + vyk * vyk
        kap2 = kap * kap
        cmap = {
            (0, 0): 1.0 + kap2 + 2.0 * a + 2.0 * cc,
            (0, 1): -a + 0.5 * m1,
            (0, -1): -a - 0.5 * m1,
            (1, 0): -cc + 0.5 * m2,
            (-1, 0): -cc - 0.5 * m2,
            (1, 1): -0.5 * bb,
            (-1, -1): -0.5 * bb,
            (1, -1): 0.5 * bb,
            (-1, 1): 0.5 * bb,
        }
        cstack = jnp.concatenate([cmap[d] for d in _D9], axis=0)
        c_ref[k] = cstack
        g = {f: None for f in _F25}
        for di, (dy, dx) in enumerate(_D9):
            s = dy * _NX + dx
            r0 = jnp.roll(cstack, -s, axis=1)
            if dx == 0:
                sh = r0
            else:
                r1 = jnp.roll(cstack, -(s - dx * _NX), axis=1)
                wrap = (xn + dx >= _NX) if dx > 0 else (xn + dx < 0)
                sh = jnp.where(wrap, r1, r0)
            for ei, (ey, ex) in enumerate(_D9):
                f = (dy + ey, dx + ex)
                term = cstack[di:di + 1, :] * sh[ei:ei + 1, :]
                g[f] = term if g[f] is None else g[f] + term
        g_ref[k] = jnp.concatenate([g[f] for f in _F25], axis=0)


def _assemble_body(g_ref, c_ref, out_ref):
    i = pl.program_id(0)
    j = pl.program_id(1)
    dij = i - j
    far = (dij != 1) & (dij != -1) & (dij != 0)

    @pl.when(far)
    def _():
        out_ref[...] = jnp.zeros_like(out_ref)

    r = jax.lax.broadcasted_iota(jnp.int32, (_NB, _NB), 0)
    c = jax.lax.broadcasted_iota(jnp.int32, (_NB, _NB), 1)
    dyy = (c // _NX - r // _NX) % _NY
    dxx = (c % _NX - r % _NX) % _NX

    @pl.when((dij == 1) | (dij == -1))
    def _():
        acc = None
        for t, (fy, fx) in enumerate(_D9):
            mask = (dyy == (fy % _NY)) & (dxx == (fx % _NX))
            v = -c_ref[0, :, t:t + 1]
            term = jnp.where(mask, v, 0.0)
            acc = term if acc is None else acc + term
        out_ref[0] = acc

    @pl.when(dij == 0)
    def _():
        interior = jnp.logical_and(i > 0, i < _NT - 1)
        add1 = jnp.where(interior, 1.0, 0.0).astype(jnp.float32)
        acc = None
        for t, (fy, fx) in enumerate(_F25):
            mask = (dyy == (fy % _NY)) & (dxx == (fx % _NX))
            v = g_ref[0, :, t:t + 1]
            if (fy, fx) == (0, 0):
                v = v + add1
            term = jnp.where(mask, v, 0.0)
            acc = term if acc is None else acc + term
        out_ref[0] = acc


def _build_q(xf, w1r, w2r, w3r, interpret=False):
    params = pl.pallas_call(
        _encoder_body,
        out_shape=jax.ShapeDtypeStruct((6 * _NT, _NB), jnp.float32),
        interpret=interpret,
    )(xf, w1r, w2r, w3r)

    def scramble(p5):
        return p5.reshape(_NT * _NB).reshape(_NB, _NT).T

    ks = scramble(params[0:_NT])
    m1s = scramble(params[_NT:2 * _NT])
    m2s = scramble(params[2 * _NT:3 * _NT])
    ga = params[3 * _NT:4 * _NT]
    vx = params[4 * _NT:5 * _NT]
    vy = params[5 * _NT:6 * _NT]

    g, cstk = pl.pallas_call(
        _coeff_body,
        out_shape=(
            jax.ShapeDtypeStruct((_NT, 25, _NB), jnp.float32),
            jax.ShapeDtypeStruct((_NT, 9, _NB), jnp.float32),
        ),
        interpret=interpret,
    )(ks, m1s, m2s, ga, vx, vy)

    gT = jnp.transpose(g, (0, 2, 1))
    cT = jnp.transpose(cstk, (0, 2, 1))

    q = pl.pallas_call(
        _assemble_body,
        grid=(_NT, _NT),
        in_specs=[
            pl.BlockSpec((1, _NB, 25), lambda i, j: (i, 0, 0)),
            pl.BlockSpec((1, _NB, 9), lambda i, j: (j, 0, 0)),
        ],
        out_specs=pl.BlockSpec((1, _NB, _NB), lambda i, j: (0, i, j)),
        out_shape=jax.ShapeDtypeStruct((1, _NT * _NB, _NT * _NB), jnp.float32),
        interpret=interpret,
    )(gT, cT)
    return q[0]


def kernel(x, kappa, m, H, W1, W2, W3):
    del kappa, m, H
    w1r = jnp.transpose(W1.reshape(W1.shape[0], W1.shape[1], 9), (2, 0, 1))
    w2r = jnp.transpose(W2.reshape(W2.shape[0], W2.shape[1], 9), (2, 0, 1))
    w3r = jnp.transpose(W3.reshape(W3.shape[0], W3.shape[1], 9), (2, 0, 1))
    qs = []
    for b in range(x.shape[0]):
        xf = x[b].reshape(_NT, _NB)
        qs.append(_build_q(xf, w1r, w2r, w3r))
    return jnp.stack(qs)


# speedup vs baseline: 3.9272x; 3.9272x over previous
import jax
import jax.numpy as jnp
from jax.experimental import pallas as pl

_NT, _NY, _NX = 5, 32, 32
_NB = _NY * _NX
_D9 = [(dy, dx) for dy in (-1, 0, 1) for dx in (-1, 0, 1)]
_F25 = [(fy, fx) for fy in (-2, -1, 0, 1, 2) for fx in (-2, -1, 0, 1, 2)]


def _softplus10(z):
    return jax.nn.softplus(10.0 * z) / 10.0


def _encoder_body(x_ref, w1_ref, w2_ref, w3_ref, out_ref):
    lane = jax.lax.broadcasted_iota(jnp.int32, (1, _NB), 1)
    yy = lane // _NX
    xx = lane % _NX

    def conv(h, w_ref):
        acc = None
        for t, (dy, dx) in enumerate(_D9):
            s = dy * _NX + dx
            rolled = h if s == 0 else jnp.roll(h, -s, axis=1)
            m = ((yy + dy >= 0) & (yy + dy < _NY)
                 & (xx + dx >= 0) & (xx + dx < _NX))
            hs = jnp.where(m, rolled, 0.0)
            p = jnp.dot(w_ref[t], hs, preferred_element_type=jnp.float32)
            acc = p if acc is None else acc + p
        return acc

    h = conv(jax.nn.relu(x_ref[...]), w1_ref)
    h = conv(jax.nn.relu(h), w2_ref)
    out_ref[...] = conv(h, w3_ref)


def _coeff_body(ks_ref, m1_ref, m2_ref, ga_ref, vx_ref, vy_ref, g_ref, c_ref):
    lane9 = jax.lax.broadcasted_iota(jnp.int32, (9, _NB), 1)
    xn = lane9 % _NX
    for k in range(_NT):
        kap = _softplus10(ks_ref[k:k + 1, :])
        gam = _softplus10(ga_ref[k:k + 1, :])
        vxk = vx_ref[k:k + 1, :]
        vyk = vy_ref[k:k + 1, :]
        m1 = m1_ref[k:k + 1, :]
        m2 = m2_ref[k:k + 1, :]
        a = gam + vxk * vxk
        bb = vxk * vyk
        cc = gam + vyk * vyk
        kap2 = kap * kap
        cmap = {
            (0, 0): 1.0 + kap2 + 2.0 * a + 2.0 * cc,
            (0, 1): -a + 0.5 * m1,
            (0, -1): -a - 0.5 * m1,
            (1, 0): -cc + 0.5 * m2,
            (-1, 0): -cc - 0.5 * m2,
            (1, 1): -0.5 * bb,
            (-1, -1): -0.5 * bb,
            (1, -1): 0.5 * bb,
            (-1, 1): 0.5 * bb,
        }
        cstack = jnp.concatenate([cmap[d] for d in _D9], axis=0)
        c_ref[k] = cstack
        g = {f: None for f in _F25}
        for di, (dy, dx) in enumerate(_D9):
            s = dy * _NX + dx
            r0 = cstack if s == 0 else jnp.roll(cstack, -s, axis=1)
            if dx == 0:
                sh = r0
            else:
                r1 = jnp.roll(cstack, -(s - dx * _NX), axis=1)
                wrap = (xn + dx >= _NX) if dx > 0 else (xn + dx < 0)
                sh = jnp.where(wrap, r1, r0)
            for ei, (ey, ex) in enumerate(_D9):
                f = (dy + ey, dx + ex)
                term = cstack[di:di + 1, :] * sh[ei:ei + 1, :]
                g[f] = term if g[f] is None else g[f] + term
        g_ref[k] = jnp.concatenate([g[f] for f in _F25], axis=0)


def _assemble_body(g_ref, c_ref, out_ref):
    i = pl.program_id(0)
    j = pl.program_id(1)
    dij = i - j
    far = (dij != 1) & (dij != -1) & (dij != 0)

    @pl.when(far)
    def _():
        out_ref[...] = jnp.zeros_like(out_ref)

    r = jax.lax.broadcasted_iota(jnp.int32, (_NB, _NB), 0)
    c = jax.lax.broadcasted_iota(jnp.int32, (_NB, _NB), 1)
    dyy = (c // _NX - r // _NX) % _NY
    dxx = (c % _NX - r % _NX) % _NX

    @pl.when((dij == 1) | (dij == -1))
    def _():
        acc = None
        for t, (fy, fx) in enumerate(_D9):
            mask = (dyy == (fy % _NY)) & (dxx == (fx % _NX))
            v = -c_ref[0, :, t:t + 1]
            term = jnp.where(mask, v, 0.0)
            acc = term if acc is None else acc + term
        out_ref[0] = acc

    @pl.when(dij == 0)
    def _():
        interior = jnp.logical_and(i > 0, i < _NT - 1)
        add1 = jnp.where(interior, 1.0, 0.0).astype(jnp.float32)
        acc = None
        for t, (fy, fx) in enumerate(_F25):
            mask = (dyy == (fy % _NY)) & (dxx == (fx % _NX))
            v = g_ref[0, :, t:t + 1]
            if (fy, fx) == (0, 0):
                v = v + add1
            term = jnp.where(mask, v, 0.0)
            acc = term if acc is None else acc + term
        out_ref[0] = acc


def _build_q(xf, w1r, w2r, w3r, interpret=False):
    params = pl.pallas_call(
        _encoder_body,
        out_shape=jax.ShapeDtypeStruct((6 * _NT, _NB), jnp.float32),
        interpret=interpret,
    )(xf, w1r, w2r, w3r)

    def scramble(p5):
        return p5.reshape(_NT * _NB).reshape(_NB, _NT).T

    ks = scramble(params[0:_NT])
    m1s = scramble(params[_NT:2 * _NT])
    m2s = scramble(params[2 * _NT:3 * _NT])
    ga = params[3 * _NT:4 * _NT]
    vx = params[4 * _NT:5 * _NT]
    vy = params[5 * _NT:6 * _NT]

    g, cstk = pl.pallas_call(
        _coeff_body,
        out_shape=(
            jax.ShapeDtypeStruct((_NT, 25, _NB), jnp.float32),
            jax.ShapeDtypeStruct((_NT, 9, _NB), jnp.float32),
        ),
        interpret=interpret,
    )(ks, m1s, m2s, ga, vx, vy)

    gT = jnp.transpose(g, (0, 2, 1))
    cT = jnp.transpose(cstk, (0, 2, 1))

    q = pl.pallas_call(
        _assemble_body,
        grid=(_NT, _NT),
        in_specs=[
            pl.BlockSpec((1, _NB, 25), lambda i, j: (i, 0, 0)),
            pl.BlockSpec((1, _NB, 9), lambda i, j: (j, 0, 0)),
        ],
        out_specs=pl.BlockSpec((1, _NB, _NB), lambda i, j: (0, i, j)),
        out_shape=jax.ShapeDtypeStruct((1, _NT * _NB, _NT * _NB), jnp.float32),
        interpret=interpret,
    )(gT, cT)
    return q[0]


def kernel(x, kappa, m, H, W1, W2, W3):
    del kappa, m, H
    w1r = jnp.transpose(W1.reshape(W1.shape[0], W1.shape[1], 9), (2, 0, 1))
    w2r = jnp.transpose(W2.reshape(W2.shape[0], W2.shape[1], 9), (2, 0, 1))
    w3r = jnp.transpose(W3.reshape(W3.shape[0], W3.shape[1], 9), (2, 0, 1))
    qs = []
    for b in range(x.shape[0]):
        xf = x[b].reshape(_NT, _NB)
        qs.append(_build_q(xf, w1r, w2r, w3r))
    return jnp.stack(qs)


# speedup vs baseline: 16.9502x; 4.3161x over previous
import jax
import jax.numpy as jnp
from jax.experimental import pallas as pl

_NT, _NY, _NX = 5, 32, 32
_NB = _NY * _NX
_D9 = [(dy, dx) for dy in (-1, 0, 1) for dx in (-1, 0, 1)]
_F25 = [(fy, fx) for fy in (-2, -1, 0, 1, 2) for fx in (-2, -1, 0, 1, 2)]


def _softplus10(z):
    return jax.nn.softplus(10.0 * z) / 10.0


def _encoder_body(x_ref, w1_ref, w2_ref, w3_ref, out_ref):
    lane = jax.lax.broadcasted_iota(jnp.int32, (1, _NB), 1)
    yy = lane // _NX
    xx = lane % _NX

    def conv(h, w_ref):
        acc = None
        for t, (dy, dx) in enumerate(_D9):
            s = dy * _NX + dx
            rolled = h if s == 0 else jnp.roll(h, -s, axis=1)
            m = ((yy + dy >= 0) & (yy + dy < _NY)
                 & (xx + dx >= 0) & (xx + dx < _NX))
            hs = jnp.where(m, rolled, 0.0)
            p = jnp.dot(w_ref[t], hs, preferred_element_type=jnp.float32)
            acc = p if acc is None else acc + p
        return acc

    h = conv(jax.nn.relu(x_ref[...]), w1_ref)
    h = conv(jax.nn.relu(h), w2_ref)
    out_ref[...] = conv(h, w3_ref)


def _coeff_body(ks_ref, m1_ref, m2_ref, ga_ref, vx_ref, vy_ref, g_ref, c_ref):
    lane9 = jax.lax.broadcasted_iota(jnp.int32, (9, _NB), 1)
    xn = lane9 % _NX
    for k in range(_NT):
        kap = _softplus10(ks_ref[k:k + 1, :])
        gam = _softplus10(ga_ref[k:k + 1, :])
        vxk = vx_ref[k:k + 1, :]
        vyk = vy_ref[k:k + 1, :]
        m1 = m1_ref[k:k + 1, :]
        m2 = m2_ref[k:k + 1, :]
        a = gam + vxk * vxk
        bb = vxk * vyk
        cc = gam + vyk * vyk
        kap2 = kap * kap
        cmap = {
            (0, 0): 1.0 + kap2 + 2.0 * a + 2.0 * cc,
            (0, 1): -a + 0.5 * m1,
            (0, -1): -a - 0.5 * m1,
            (1, 0): -cc + 0.5 * m2,
            (-1, 0): -cc - 0.5 * m2,
            (1, 1): -0.5 * bb,
            (-1, -1): -0.5 * bb,
            (1, -1): 0.5 * bb,
            (-1, 1): 0.5 * bb,
        }
        cstack = jnp.concatenate([cmap[d] for d in _D9], axis=0)
        c_ref[k] = cstack
        g = {f: None for f in _F25}
        for di, (dy, dx) in enumerate(_D9):
            s = dy * _NX + dx
            r0 = cstack if s == 0 else jnp.roll(cstack, -s, axis=1)
            if dx == 0:
                sh = r0
            else:
                r1 = jnp.roll(cstack, -(s - dx * _NX), axis=1)
                wrap = (xn + dx >= _NX) if dx > 0 else (xn + dx < 0)
                sh = jnp.where(wrap, r1, r0)
            for ei, (ey, ex) in enumerate(_D9):
                f = (dy + ey, dx + ex)
                term = cstack[di:di + 1, :] * sh[ei:ei + 1, :]
                g[f] = term if g[f] is None else g[f] + term
        g_ref[k] = jnp.concatenate([g[f] for f in _F25], axis=0)


def _strip_segments(fys):
    segs = []
    for ry in range(_NY):
        cyts = [(ry + fy) % _NY for fy in fys]
        runs = []
        k0 = 0
        for k in range(1, len(cyts) + 1):
            if k == len(cyts) or cyts[k] != cyts[k - 1] + 1:
                runs.append((k0 * _NX, cyts[k0] * _NX, (k - k0) * _NX))
                k0 = k
        segs.append(runs)
    return segs


_SEG5 = _strip_segments(range(-2, 3))
_SEG3 = _strip_segments(range(-1, 2))


def _assemble_body(g_ref, c_ref, out_ref):
    i = pl.program_id(0)
    j = pl.program_id(1)
    dij = i - j

    out_ref[...] = jnp.zeros_like(out_ref)

    rx = jax.lax.broadcasted_iota(jnp.int32, (_NB, 1), 0) % _NX
    lx = jax.lax.broadcasted_iota(jnp.int32, (_NB, _NX), 1)
    dxm = (lx - rx) % _NX

    def strips(fys, fxs, value):
        pieces = []
        for fy in fys:
            acc = None
            for fx in fxs:
                term = jnp.where(dxm == (fx % _NX), value(fy, fx), 0.0)
                acc = term if acc is None else acc + term
            pieces.append(acc)
        return jnp.concatenate(pieces, axis=1)

    @pl.when((dij == 1) | (dij == -1))
    def _():
        w = strips(range(-1, 2), range(-1, 2),
                   lambda fy, fx: -c_ref[0, :, (fy + 1) * 3 + (fx + 1):
                                         (fy + 1) * 3 + (fx + 1) + 1])
        for ry in range(_NY):
            rr = pl.ds(ry * _NX, _NX)
            for (w0, b0, wd) in _SEG3[ry]:
                out_ref[0, rr, pl.ds(b0, wd)] = w[ry * _NX:(ry + 1) * _NX,
                                                  w0:w0 + wd]

    @pl.when(dij == 0)
    def _():
        interior = jnp.logical_and(i > 0, i < _NT - 1)
        add1 = jnp.where(interior, 1.0, 0.0).astype(jnp.float32)

        def val(fy, fx):
            t = (fy + 2) * 5 + (fx + 2)
            v = g_ref[0, :, t:t + 1]
            if (fy, fx) == (0, 0):
                v = v + add1
            return v

        w = strips(range(-2, 3), range(-2, 3), val)
        for ry in range(_NY):
            rr = pl.ds(ry * _NX, _NX)
            for (w0, b0, wd) in _SEG5[ry]:
                out_ref[0, rr, pl.ds(b0, wd)] = w[ry * _NX:(ry + 1) * _NX,
                                                  w0:w0 + wd]


def _build_q(xf, w1r, w2r, w3r, interpret=False):
    params = pl.pallas_call(
        _encoder_body,
        out_shape=jax.ShapeDtypeStruct((6 * _NT, _NB), jnp.float32),
        interpret=interpret,
    )(xf, w1r, w2r, w3r)

    def scramble(p5):
        return p5.reshape(_NT * _NB).reshape(_NB, _NT).T

    ks = scramble(params[0:_NT])
    m1s = scramble(params[_NT:2 * _NT])
    m2s = scramble(params[2 * _NT:3 * _NT])
    ga = params[3 * _NT:4 * _NT]
    vx = params[4 * _NT:5 * _NT]
    vy = params[5 * _NT:6 * _NT]

    g, cstk = pl.pallas_call(
        _coeff_body,
        out_shape=(
            jax.ShapeDtypeStruct((_NT, 25, _NB), jnp.float32),
            jax.ShapeDtypeStruct((_NT, 9, _NB), jnp.float32),
        ),
        interpret=interpret,
    )(ks, m1s, m2s, ga, vx, vy)

    gT = jnp.transpose(g, (0, 2, 1))
    cT = jnp.transpose(cstk, (0, 2, 1))

    q = pl.pallas_call(
        _assemble_body,
        grid=(_NT, _NT),
        in_specs=[
            pl.BlockSpec((1, _NB, 25), lambda i, j: (i, 0, 0)),
            pl.BlockSpec((1, _NB, 9), lambda i, j: (j, 0, 0)),
        ],
        out_specs=pl.BlockSpec((1, _NB, _NB), lambda i, j: (0, i, j)),
        out_shape=jax.ShapeDtypeStruct((1, _NT * _NB, _NT * _NB), jnp.float32),
        interpret=interpret,
    )(gT, cT)
    return q[0]


def kernel(x, kappa, m, H, W1, W2, W3):
    del kappa, m, H
    w1r = jnp.transpose(W1.reshape(W1.shape[0], W1.shape[1], 9), (2, 0, 1))
    w2r = jnp.transpose(W2.reshape(W2.shape[0], W2.shape[1], 9), (2, 0, 1))
    w3r = jnp.transpose(W3.reshape(W3.shape[0], W3.shape[1], 9), (2, 0, 1))
    qs = []
    for b in range(x.shape[0]):
        xf = x[b].reshape(_NT, _NB)
        qs.append(_build_q(xf, w1r, w2r, w3r))
    return jnp.stack(qs)
